# Initial kernel scaffold; baseline (speedup 1.0000x reference)
import jax
import jax.numpy as jnp
from jax.experimental import pallas as pl


def kernel(input_ids, non_tf_mask, padding_mask, u, v, prior_pair_keys):
    raise NotImplementedError("write your pallas kernel here")


# speedup vs baseline: 3411.4613x; 3411.4613x over previous
import functools

import jax
import jax.numpy as jnp
from jax import lax
from jax.experimental import pallas as pl
from jax.experimental.pallas import tpu as pltpu
from jax.experimental.pallas import tpu_sc as plsc

_PAD_INDEX = 0
_PAIR_KEY_BASE = 20000
_NUM_GENES = 20000
_ROW_WORDS = 640
_B = 4
_S = 2048
_K = 64
_ROWS = _B * _S
_NW = 32
_RPW = _ROWS // _NW
_CHUNK = 16
_TC_BLK = 256


def _sc_mask_kernel(bitmap_hbm, ids_hbm, out_hbm, a_vec, bcol_w, bcol_bit,
                    row_buf, out_buf):
    wid = lax.axis_index("s") * 2 + lax.axis_index("c")
    base = wid * _RPW
    pltpu.sync_copy(ids_hbm.at[pl.ds(base, _RPW)], a_vec.at[pl.ds(0, _RPW)])
    batch_base = (wid // (_S // _RPW)) * _S
    pltpu.sync_copy(ids_hbm.at[pl.ds(batch_base, _S)], bcol_w)

    def col_prep(k, _):
        x = bcol_w[pl.ds(k * 16, 16)]
        bcol_bit[pl.ds(k * 16, 16)] = jnp.bitwise_and(x, 31)
        bcol_w[pl.ds(k * 16, 16)] = lax.shift_right_logical(x, 5)
        return 0

    lax.fori_loop(0, _S // 16, col_prep, 0)

    def row_body(r, _):
        a_s = a_vec[pl.ds(r, 16)][0]
        pltpu.sync_copy(bitmap_hbm.at[pl.ds(a_s * _ROW_WORDS, _ROW_WORDS)],
                        row_buf)

        def col_body(k, _):
            off = k * 16
            w16 = bcol_w[pl.ds(off, 16)]
            b16 = bcol_bit[pl.ds(off, 16)]
            word = plsc.load_gather(row_buf, [w16])
            m = jnp.bitwise_and(lax.shift_right_logical(word, b16), 1)
            out_buf[pl.ds(off, 16)] = m.astype(jnp.float32)
            return 0

        lax.fori_loop(0, _S // 16, col_body, 0)
        pltpu.sync_copy(out_buf, out_hbm.at[pl.ds((base + r) * _S, _S)])
        return 0

    lax.fori_loop(0, _RPW, row_body, 0)


def _sc_mask(bitmap, ids_flat):
    mesh = plsc.VectorSubcoreMesh(core_axis_name="c", subcore_axis_name="s")
    run = functools.partial(
        pl.kernel,
        mesh=mesh,
        compiler_params=pltpu.CompilerParams(needs_layout_passes=False),
        out_type=jax.ShapeDtypeStruct((_ROWS * _S,), jnp.float32),
        scratch_types=[
            pltpu.VMEM((_RPW + 16,), jnp.int32),
            pltpu.VMEM((_S,), jnp.int32),
            pltpu.VMEM((_S,), jnp.int32),
            pltpu.VMEM((_ROW_WORDS,), jnp.int32),
            pltpu.VMEM((_S,), jnp.float32),
        ],
    )(_sc_mask_kernel)
    return run(bitmap.reshape(-1), ids_flat)


def _tc_loss_kernel(u_ref, v_ref, m_ref, tf_ref, act_ref, num_ref, den_ref):
    b = pl.program_id(0)
    i = pl.program_id(1)

    @pl.when(jnp.logical_and(b == 0, i == 0))
    def _():
        num_ref[...] = jnp.zeros((1, 1), jnp.float32)
        den_ref[...] = jnp.zeros((1, 1), jnp.float32)

    u = u_ref[0]
    v = v_ref[0]
    p = lax.dot_general(u, v, (((1,), (1,)), ((), ())),
                        preferred_element_type=jnp.float32)
    p = jnp.clip(p, 1e-8, 1.0 - 1e-8)
    m = m_ref[0]
    loss = -(m * jnp.log(p) + (1.0 - m) * jnp.log(1.0 - p))
    tf = tf_ref[0, 0]
    act = act_ref[0, 0]
    sup = tf[:, None] * act[None, :]
    num_ref[...] += jnp.sum(loss * sup).reshape(1, 1)
    den_ref[...] += (jnp.sum(tf) * jnp.sum(act)).reshape(1, 1)


def _tc_loss(u, v, mask, tf_f, act_f):
    nblk = _S // _TC_BLK
    grid = (_B, nblk)
    num, den = pl.pallas_call(
        _tc_loss_kernel,
        grid=grid,
        in_specs=[
            pl.BlockSpec((1, _TC_BLK, _K), lambda b, i: (b, i, 0)),
            pl.BlockSpec((1, _S, _K), lambda b, i: (b, 0, 0)),
            pl.BlockSpec((1, _TC_BLK, _S), lambda b, i: (b, i, 0)),
            pl.BlockSpec((1, 1, _TC_BLK), lambda b, i: (b * nblk + i, 0, 0)),
            pl.BlockSpec((1, 1, _S), lambda b, i: (b, 0, 0)),
        ],
        out_specs=[
            pl.BlockSpec((1, 1), lambda b, i: (0, 0)),
            pl.BlockSpec((1, 1), lambda b, i: (0, 0)),
        ],
        out_shape=[
            jax.ShapeDtypeStruct((1, 1), jnp.float32),
            jax.ShapeDtypeStruct((1, 1), jnp.float32),
        ],
    )(u, v, mask, tf_f, act_f)
    return num[0, 0], den[0, 0]


def _build_bitmap(prior_pair_keys):
    sk = jnp.sort(prior_pair_keys.astype(jnp.int32))
    keep = jnp.concatenate(
        [jnp.ones((1,), jnp.bool_), sk[1:] != sk[:-1]])
    a = sk // _PAIR_KEY_BASE
    bcol = sk % _PAIR_KEY_BASE
    w = lax.shift_right_logical(bcol, 5)
    bit = jnp.bitwise_and(bcol, 31)
    vals = jnp.where(keep, lax.shift_left(jnp.int32(1), bit), jnp.int32(0))
    bitmap = jnp.zeros((_NUM_GENES, _ROW_WORDS), jnp.int32)
    return bitmap.at[a, w].add(vals, mode="drop")


def kernel(input_ids, non_tf_mask, padding_mask, u, v, prior_pair_keys):
    ids = input_ids.astype(jnp.int32)
    active = jnp.logical_and(ids != _PAD_INDEX, jnp.logical_not(padding_mask))
    tf = jnp.logical_and(active, jnp.logical_not(non_tf_mask))
    act_f = active.astype(jnp.float32).reshape(_B, 1, _S)
    tf_f = tf.astype(jnp.float32).reshape(_B * (_S // _TC_BLK), 1, _TC_BLK)

    bitmap = _build_bitmap(prior_pair_keys)
    mask_flat = _sc_mask(bitmap, ids.reshape(_ROWS))
    mask = mask_flat.reshape(_B, _S, _S)

    num, den = _tc_loss(u, v, mask, tf_f, act_f)
    return num / (den + 1e-8)
